# Initial kernel scaffold; baseline (speedup 1.0000x reference)
import jax
import jax.numpy as jnp
from jax.experimental import pallas as pl


def kernel(indices, emb_table, gate_table):
    raise NotImplementedError("write your pallas kernel here")


# speedup vs baseline: 1.0882x; 1.0882x over previous
import functools

import jax
import jax.numpy as jnp
from jax import lax
from jax.experimental import pallas as pl
from jax.experimental.pallas import tpu as pltpu
from jax.experimental.pallas import tpu_sc as plsc

NC, NS, LANES = 2, 16, 16
NW = NC * NS
CHUNK = 128


def _gate_func(x):
    L = 1000000.0
    x_ = x - 0.5
    b = jnp.where(x >= 0.5, 1.0, 0.0).astype(jnp.float32)
    t = L * x_
    ti = t.astype(jnp.int32).astype(jnp.float32)
    ft = jnp.where(ti > t, ti - 1.0, ti)
    return b + (t - ft) / L


def _make_sc_kernel(n_chunks: int, d: int):
    per_w = n_chunks * CHUNK

    @functools.partial(
        pl.kernel,
        out_type=jax.ShapeDtypeStruct((NW * per_w, d), jnp.float32),
        mesh=plsc.VectorSubcoreMesh(
            core_axis_name="c", subcore_axis_name="s",
            num_cores=NC, num_subcores=NS),
        scratch_types=[
            pltpu.VMEM((n_chunks, CHUNK), jnp.int32),
            pltpu.VMEM((CHUNK, d), jnp.float32),
            pltpu.VMEM((CHUNK, d), jnp.float32),
            pltpu.VMEM((CHUNK, d), jnp.float32),
            pltpu.VMEM((CHUNK, d), jnp.float32),
            pltpu.SemaphoreType.DMA,
            pltpu.SemaphoreType.DMA,
            pltpu.SemaphoreType.DMA,
            pltpu.SemaphoreType.DMA,
        ],
        compiler_params=pltpu.CompilerParams(use_tc_tiling_on_sc=False),
    )
    def k(idx_hbm, emb_hbm, gate_hbm, out_hbm,
          idx_v, emb0, emb1, gate0, gate1, gs0, gs1, os0, os1):
        wid = lax.axis_index("s") * NC + lax.axis_index("c")
        base = wid * per_w
        ebuf = (emb0, emb1)
        gbuf = (gate0, gate1)
        gsem = (gs0, gs1)
        osem = (os0, os1)

        pltpu.sync_copy(idx_hbm.at[wid], idx_v)

        def fire_gathers(chunk, slot):
            row = idx_v.at[chunk]
            pltpu.async_copy(emb_hbm.at[row], ebuf[slot], gsem[slot])
            pltpu.async_copy(gate_hbm.at[row], gbuf[slot], gsem[slot])

        def wait_gathers(chunk, slot):
            row = idx_v.at[chunk]
            pltpu.make_async_copy(emb_hbm.at[row], ebuf[slot], gsem[slot]).wait()
            pltpu.make_async_copy(gate_hbm.at[row], gbuf[slot], gsem[slot]).wait()

        def out_slice(chunk):
            return out_hbm.at[pl.ds(base + chunk * CHUNK, CHUNK)]

        def compute_chunk(slot):
            e, g = ebuf[slot], gbuf[slot]

            def row_body(r, carry):
                for h in range(0, d, LANES):
                    sl = (r, pl.ds(h, LANES))
                    e[sl] = e[sl] * _gate_func(g[sl])
                return carry

            lax.fori_loop(0, CHUNK, row_body, 0, unroll=2)

        fire_gathers(0, 0)

        def loop_body(j, carry):
            for b in range(2):
                chunk = j + b
                nxt = chunk + 1
                nslot = 1 - b

                @pl.when(nxt < n_chunks)
                def _():
                    @pl.when(chunk >= 1)
                    def _():
                        pltpu.make_async_copy(
                            ebuf[nslot], out_slice(chunk - 1), osem[nslot]
                        ).wait()
                    fire_gathers(nxt, nslot)

                wait_gathers(chunk, b)
                compute_chunk(b)
                pltpu.async_copy(ebuf[b], out_slice(chunk), osem[b])
            return carry

        lax.fori_loop(0, n_chunks // 2, lambda i, c: loop_body(2 * i, c), 0)

        pltpu.make_async_copy(ebuf[0], out_slice(n_chunks - 2), osem[0]).wait()
        pltpu.make_async_copy(ebuf[1], out_slice(n_chunks - 1), osem[1]).wait()

    return k


def kernel(indices, emb_table, gate_table):
    b, f = indices.shape
    v, d = emb_table.shape
    n = b * f
    assert n % (NW * CHUNK) == 0 and d % LANES == 0
    n_chunks = n // (NW * CHUNK)
    idx = indices.astype(jnp.int32).reshape(NW, n_chunks, CHUNK)
    out = _make_sc_kernel(n_chunks, d)(idx, emb_table, gate_table)
    return out.reshape(b, f, d)


# speedup vs baseline: 1.3647x; 1.2541x over previous
import functools

import jax
import jax.numpy as jnp
from jax import lax
from jax.experimental import pallas as pl

from jax.experimental.pallas import tpu as pltpu
from jax.experimental.pallas import tpu_sc as plsc

NC, NS, LANES = 2, 16, 16
NW = NC * NS
IROW = 128
SUB = 4
CHUNK = IROW * SUB


def _make_sc_kernel(n_chunks: int, d: int):
    per_w = n_chunks * CHUNK

    @functools.partial(
        pl.kernel,
        out_type=jax.ShapeDtypeStruct((NW * per_w, d), jnp.float32),
        mesh=plsc.VectorSubcoreMesh(
            core_axis_name="c", subcore_axis_name="s",
            num_cores=NC, num_subcores=NS),
        scratch_types=[
            pltpu.VMEM((n_chunks * SUB, IROW), jnp.int32),
            pltpu.VMEM((CHUNK, d), jnp.float32),
            pltpu.VMEM((CHUNK, d), jnp.float32),
            pltpu.VMEM((CHUNK, d), jnp.float32),
            pltpu.VMEM((CHUNK, d), jnp.float32),
            pltpu.SemaphoreType.DMA,
            pltpu.SemaphoreType.DMA,
            pltpu.SemaphoreType.DMA,
            pltpu.SemaphoreType.DMA,
        ],
        compiler_params=pltpu.CompilerParams(use_tc_tiling_on_sc=False),
    )
    def k(idx_hbm, emb_hbm, gate_hbm, out_hbm,
          idx_v, emb0, emb1, gate0, gate1, gs0, gs1, os0, os1):
        wid = lax.axis_index("s") * NC + lax.axis_index("c")
        base = wid * per_w
        ebuf = (emb0, emb1)
        gbuf = (gate0, gate1)
        gsem = (gs0, gs1)
        osem = (os0, os1)

        pltpu.sync_copy(idx_hbm.at[wid], idx_v)

        def fire_gathers(chunk, slot):
            for q in range(SUB):
                row = idx_v.at[chunk * SUB + q]
                dst = pl.ds(q * IROW, IROW)
                pltpu.async_copy(emb_hbm.at[row], ebuf[slot].at[dst], gsem[slot])
                pltpu.async_copy(gate_hbm.at[row], gbuf[slot].at[dst], gsem[slot])

        def wait_gathers(slot):
            row = idx_v.at[0]
            pltpu.make_async_copy(emb_hbm.at[row], ebuf[slot], gsem[slot]).wait()
            pltpu.make_async_copy(gate_hbm.at[row], gbuf[slot], gsem[slot]).wait()

        def out_slice(chunk):
            return out_hbm.at[pl.ds(base + chunk * CHUNK, CHUNK)]

        def compute_chunk(slot):
            e, g = ebuf[slot], gbuf[slot]

            def row_body(r, carry):
                for h in range(0, d, LANES):
                    sl = (r, pl.ds(h, LANES))
                    e[sl] = jnp.where(g[sl] >= 0.5, e[sl], 0.0)
                return carry

            lax.fori_loop(0, CHUNK, row_body, 0, unroll=8)

        fire_gathers(0, 0)

        def loop_body(j, carry):
            for b in range(2):
                chunk = j + b
                nxt = chunk + 1
                nslot = 1 - b

                @pl.when(nxt < n_chunks)
                def _():
                    @pl.when(chunk >= 1)
                    def _():
                        pltpu.make_async_copy(
                            ebuf[nslot], out_slice(chunk - 1), osem[nslot]
                        ).wait()
                    fire_gathers(nxt, nslot)

                wait_gathers(b)
                compute_chunk(b)
                pltpu.async_copy(ebuf[b], out_slice(chunk), osem[b])
            return carry

        lax.fori_loop(0, n_chunks // 2, lambda i, c: loop_body(2 * i, c), 0)

        pltpu.make_async_copy(ebuf[0], out_slice(n_chunks - 2), osem[0]).wait()
        pltpu.make_async_copy(ebuf[1], out_slice(n_chunks - 1), osem[1]).wait()

    return k


def kernel(indices, emb_table, gate_table):
    b, f = indices.shape
    v, d = emb_table.shape
    n = b * f
    assert n % (NW * CHUNK) == 0 and d % LANES == 0
    n_chunks = n // (NW * CHUNK)
    idx = indices.astype(jnp.int32).reshape(NW, n_chunks * SUB, IROW)
    out = _make_sc_kernel(n_chunks, d)(idx, emb_table, gate_table)
    return out.reshape(b, f, d)
